# Initial kernel scaffold; baseline (speedup 1.0000x reference)
import functools

import jax
import jax.numpy as jnp
from jax import lax
from jax.experimental import pallas as pl
from jax.experimental.pallas import tpu as pltpu
from jax.experimental.pallas import tpu_sc as plsc

N = 10000
E = 320000
D = 128
C = 128

NUM_CORES = 2
NUM_SUBCORES = 16
NW = NUM_CORES * NUM_SUBCORES
CPW = D // NW
CHUNK = 4000
LANES = 16


def _tr_body(x_ref, o_ref):
    o_ref[...] = x_ref[...].T


def _transpose_tc(x):
    BN = 1000
    return pl.pallas_call(
        _tr_body,
        grid=(N // BN,),
        in_specs=[pl.BlockSpec((BN, D), lambda i: (i, 0))],
        out_specs=pl.BlockSpec((D, BN), lambda i: (0, i)),
        out_shape=jax.ShapeDtypeStruct((D, N), jnp.float32),
    )(x)


def _sc_agg_body(xT_hbm, ei_hbm, ew_hbm, agg_hbm, xrows, acc, src_v, dst_v, ew_v):
    wid = lax.axis_index("s") * NUM_CORES + lax.axis_index("c")
    c0 = wid * CPW

    pltpu.sync_copy(xT_hbm.at[pl.ds(c0, CPW)], xrows)

    zero = jnp.zeros((LANES,), jnp.float32)

    def zbody(i, carry):
        for c in range(CPW):
            acc[c, pl.ds(i * LANES, LANES)] = zero
        return carry

    lax.fori_loop(0, N // LANES, zbody, 0)

    cvecs = [jnp.full((LANES,), c, jnp.int32) for c in range(CPW)]

    def chunk_body(k, carry):
        off = k * CHUNK
        pltpu.sync_copy(ei_hbm.at[0, pl.ds(off, CHUNK)], src_v)
        pltpu.sync_copy(ei_hbm.at[1, pl.ds(off, CHUNK)], dst_v)
        pltpu.sync_copy(ew_hbm.at[pl.ds(off, CHUNK)], ew_v)

        def gbody(g, gcarry):
            base = g * LANES
            s = src_v[pl.ds(base, LANES)]
            dd = dst_v[pl.ds(base, LANES)]
            w = ew_v[pl.ds(base, LANES)]
            for c in range(CPW):
                v = plsc.load_gather(xrows, [cvecs[c], s])
                plsc.addupdate_scatter(acc, [cvecs[c], dd], v * w)
            return gcarry

        lax.fori_loop(0, CHUNK // LANES, gbody, 0)
        return carry

    lax.fori_loop(0, E // CHUNK, chunk_body, 0)

    pltpu.sync_copy(acc, agg_hbm.at[pl.ds(c0, CPW)])


def _sc_agg(xT, edge_index, edge_weight):
    mesh = plsc.VectorSubcoreMesh(core_axis_name="c", subcore_axis_name="s")
    f = functools.partial(
        pl.kernel,
        mesh=mesh,
        out_type=jax.ShapeDtypeStruct((D, N), jnp.float32),
        scratch_types=[
            pltpu.VMEM((CPW, N), jnp.float32),
            pltpu.VMEM((CPW, N), jnp.float32),
            pltpu.VMEM((CHUNK,), jnp.int32),
            pltpu.VMEM((CHUNK,), jnp.int32),
            pltpu.VMEM((CHUNK,), jnp.float32),
        ],
    )(_sc_agg_body)
    return f(xT, edge_index, edge_weight)


def _mm_body(a_ref, w_ref, b_ref, o_ref):
    o_ref[...] = lax.dot_general(
        a_ref[...], w_ref[...], (((0,), (0,)), ((), ())),
        preferred_element_type=jnp.float32,
    ) + b_ref[...]


def _matmul_tc(aggT, W, b2):
    BN = 1000
    return pl.pallas_call(
        _mm_body,
        grid=(N // BN,),
        in_specs=[
            pl.BlockSpec((D, BN), lambda i: (0, i)),
            pl.BlockSpec((D, C), lambda i: (0, 0)),
            pl.BlockSpec((1, C), lambda i: (0, 0)),
        ],
        out_specs=pl.BlockSpec((BN, C), lambda i: (i, 0)),
        out_shape=jax.ShapeDtypeStruct((N, C), jnp.float32),
    )(aggT, W, b2)


def kernel(x, edge_index, edge_weight, W, b):
    xT = _transpose_tc(x)
    aggT = _sc_agg(xT, edge_index, edge_weight)
    return _matmul_tc(aggT, W, b[None, :])


# speedup vs baseline: 2.2513x; 2.2513x over previous
import functools

import jax
import jax.numpy as jnp
from jax import lax
from jax.experimental import pallas as pl
from jax.experimental.pallas import tpu as pltpu
from jax.experimental.pallas import tpu_sc as plsc

N = 10000
E = 320000
D = 128
C = 128

NUM_CORES = 2
NUM_SUBCORES = 16
NW = NUM_CORES * NUM_SUBCORES
CPW = D // NW
CHUNK = 4000
LANES = 16


def _tr_body(x_ref, o_ref):
    o_ref[...] = x_ref[...].T


def _transpose_tc(x):
    return pl.pallas_call(
        _tr_body,
        out_shape=jax.ShapeDtypeStruct((D, N), jnp.float32),
    )(x)


def _sc_agg_body(xT_hbm, src_hbm, dst_hbm, ew_hbm, agg_hbm, xrows, acc, src_v, dst_v, ew_v):
    wid = lax.axis_index("s") * NUM_CORES + lax.axis_index("c")
    c0 = wid * CPW

    pltpu.sync_copy(xT_hbm.at[pl.ds(c0 * N, CPW * N)], xrows)

    zero = jnp.zeros((LANES,), jnp.float32)

    def zbody(i, carry):
        acc[pl.ds(i * LANES, LANES)] = zero
        return carry

    lax.fori_loop(0, (CPW * N) // LANES, zbody, 0)

    def chunk_body(k, carry):
        off = k * CHUNK
        pltpu.sync_copy(src_hbm.at[pl.ds(off, CHUNK)], src_v)
        pltpu.sync_copy(dst_hbm.at[pl.ds(off, CHUNK)], dst_v)
        pltpu.sync_copy(ew_hbm.at[pl.ds(off, CHUNK)], ew_v)

        def gbody(g, gcarry):
            base = g * LANES
            s = src_v[pl.ds(base, LANES)]
            dd = dst_v[pl.ds(base, LANES)]
            w = ew_v[pl.ds(base, LANES)]
            for c in range(CPW):
                v = plsc.load_gather(xrows, [s + (c * N)])
                plsc.addupdate_scatter(acc, [dd + (c * N)], v * w)
            return gcarry

        lax.fori_loop(0, CHUNK // LANES, gbody, 0)
        return carry

    lax.fori_loop(0, E // CHUNK, chunk_body, 0)

    pltpu.sync_copy(acc, agg_hbm.at[pl.ds(c0 * N, CPW * N)])


def _sc_agg(xT, src, dst, edge_weight):
    mesh = plsc.VectorSubcoreMesh(core_axis_name="c", subcore_axis_name="s")
    f = functools.partial(
        pl.kernel,
        mesh=mesh,
        compiler_params=pltpu.CompilerParams(needs_layout_passes=False),
        out_type=jax.ShapeDtypeStruct((D * N,), jnp.float32),
        scratch_types=[
            pltpu.VMEM((CPW * N,), jnp.float32),
            pltpu.VMEM((CPW * N,), jnp.float32),
            pltpu.VMEM((CHUNK,), jnp.int32),
            pltpu.VMEM((CHUNK,), jnp.int32),
            pltpu.VMEM((CHUNK,), jnp.float32),
        ],
    )(_sc_agg_body)
    return f(xT.reshape(-1), src, dst, edge_weight).reshape(D, N)


def _mm_body(a_ref, w_ref, b_ref, o_ref):
    o_ref[...] = lax.dot_general(
        a_ref[...], w_ref[...], (((0,), (0,)), ((), ())),
        preferred_element_type=jnp.float32,
    ) + b_ref[...]


def _matmul_tc(aggT, W, b2):
    return pl.pallas_call(
        _mm_body,
        out_shape=jax.ShapeDtypeStruct((N, C), jnp.float32),
    )(aggT, W, b2)


def kernel(x, edge_index, edge_weight, W, b):
    xT = _transpose_tc(x)
    aggT = _sc_agg(xT, edge_index[0], edge_index[1], edge_weight)
    return _matmul_tc(aggT, W, b[None, :])


# speedup vs baseline: 2.9174x; 1.2958x over previous
import functools

import jax
import jax.numpy as jnp
from jax import lax
from jax.experimental import pallas as pl
from jax.experimental.pallas import tpu as pltpu
from jax.experimental.pallas import tpu_sc as plsc

N = 10000
E = 320000
D = 128
C = 128

NUM_CORES = 2
NUM_SUBCORES = 16
NW = NUM_CORES * NUM_SUBCORES
CPW = D // NW
CHUNK = 6400
NCH = E // CHUNK
U = 8
LANES = 16


def _tr_body(x_ref, o_ref):
    o_ref[...] = x_ref[...].T


def _transpose_tc(x):
    return pl.pallas_call(
        _tr_body,
        out_shape=jax.ShapeDtypeStruct((D, N), jnp.float32),
    )(x)


def _sc_agg_body(xT_hbm, src_hbm, dst_hbm, ew_hbm, agg_hbm, xrows, acc, src_v,
                 dst_v, ew_v, sem, xsem):
    wid = lax.axis_index("s") * NUM_CORES + lax.axis_index("c")
    c0 = wid * CPW

    xcopy = pltpu.make_async_copy(xT_hbm.at[pl.ds(c0 * N, CPW * N)], xrows, xsem)
    xcopy.start()

    zero = jnp.zeros((LANES,), jnp.float32)
    ZU = 10

    def zbody(i, carry):
        for u in range(ZU):
            acc[pl.ds((i * ZU + u) * LANES, LANES)] = zero
        return carry

    lax.fori_loop(0, (CPW * N) // (LANES * ZU), zbody, 0)

    def _edge_copies(i, buf):
        off = i * CHUNK
        voff = buf * CHUNK
        return (
            pltpu.make_async_copy(
                src_hbm.at[pl.ds(off, CHUNK)], src_v.at[pl.ds(voff, CHUNK)], sem),
            pltpu.make_async_copy(
                dst_hbm.at[pl.ds(off, CHUNK)], dst_v.at[pl.ds(voff, CHUNK)], sem),
            pltpu.make_async_copy(
                ew_hbm.at[pl.ds(off, CHUNK)], ew_v.at[pl.ds(voff, CHUNK)], sem),
        )

    def _issue(i, buf):
        for cp in _edge_copies(i, buf):
            cp.start()

    _issue(0, 0)
    xcopy.wait()

    def chunk_body(i, carry):
        buf = jnp.bitwise_and(i, 1)
        voff = buf * CHUNK
        for cp in _edge_copies(i, buf):
            cp.wait()

        @pl.when(i + 1 < NCH)
        def _prefetch():
            _issue(i + 1, 1 - buf)

        def gbody(g, gcarry):
            base0 = voff + g * (LANES * U)
            for u in range(U):
                base = base0 + u * LANES
                s = src_v[pl.ds(base, LANES)]
                dd = dst_v[pl.ds(base, LANES)]
                w = ew_v[pl.ds(base, LANES)]
                for c in range(CPW):
                    v = plsc.load_gather(xrows, [s + (c * N)])
                    plsc.addupdate_scatter(acc, [dd + (c * N)], v * w)
            return gcarry

        lax.fori_loop(0, CHUNK // (LANES * U), gbody, 0)
        return carry

    lax.fori_loop(0, NCH, chunk_body, 0)

    pltpu.sync_copy(acc, agg_hbm.at[pl.ds(c0 * N, CPW * N)])


def _sc_agg(xT, src, dst, edge_weight):
    mesh = plsc.VectorSubcoreMesh(core_axis_name="c", subcore_axis_name="s")
    f = functools.partial(
        pl.kernel,
        mesh=mesh,
        compiler_params=pltpu.CompilerParams(needs_layout_passes=False),
        out_type=jax.ShapeDtypeStruct((D * N,), jnp.float32),
        scratch_types=[
            pltpu.VMEM((CPW * N,), jnp.float32),
            pltpu.VMEM((CPW * N,), jnp.float32),
            pltpu.VMEM((2 * CHUNK,), jnp.int32),
            pltpu.VMEM((2 * CHUNK,), jnp.int32),
            pltpu.VMEM((2 * CHUNK,), jnp.float32),
            pltpu.SemaphoreType.DMA,
            pltpu.SemaphoreType.DMA,
        ],
    )(_sc_agg_body)
    return f(xT.reshape(-1), src, dst, edge_weight).reshape(D, N)


def _mm_body(a_ref, w_ref, b_ref, o_ref):
    o_ref[...] = lax.dot_general(
        a_ref[...], w_ref[...], (((0,), (0,)), ((), ())),
        preferred_element_type=jnp.float32,
    ) + b_ref[...]


def _matmul_tc(aggT, W, b2):
    return pl.pallas_call(
        _mm_body,
        out_shape=jax.ShapeDtypeStruct((N, C), jnp.float32),
    )(aggT, W, b2)


def kernel(x, edge_index, edge_weight, W, b):
    xT = _transpose_tc(x)
    aggT = _sc_agg(xT, edge_index[0], edge_index[1], edge_weight)
    return _matmul_tc(aggT, W, b[None, :])


# speedup vs baseline: 7.1235x; 2.4418x over previous
import functools

import jax
import jax.numpy as jnp
from jax import lax
from jax.experimental import pallas as pl
from jax.experimental.pallas import tpu as pltpu
from jax.experimental.pallas import tpu_sc as plsc

N = 10000
E = 320000
D = 128
C = 128

NUM_CORES = 2
NUM_SUBCORES = 16
NW = NUM_CORES * NUM_SUBCORES
CPW = D // NW
CHUNK = 6400
NCH = E // CHUNK
U = 8
LANES = 16


def _tr_body(x_ref, o_ref):
    o_ref[...] = x_ref[...].T


def _transpose_tc(x):
    return pl.pallas_call(
        _tr_body,
        out_shape=jax.ShapeDtypeStruct((D, N), jnp.float32),
    )(x)


def _sc_agg_body(xT_hbm, src_hbm, dst_hbm, ew_hbm, agg_hbm, x0, x1, x2, x3,
                 a0, a1, a2, a3, src_v, dst_v, ew_v, sem, xsem):
    wid = lax.axis_index("s") * NUM_CORES + lax.axis_index("c")
    c0 = wid * CPW
    xrefs = (x0, x1, x2, x3)
    arefs = (a0, a1, a2, a3)

    xcopies = [
        pltpu.make_async_copy(xT_hbm.at[pl.ds((c0 + c) * N, N)], xrefs[c], xsem)
        for c in range(CPW)
    ]
    for cp in xcopies:
        cp.start()

    zero = jnp.zeros((LANES,), jnp.float32)
    ZU = 5

    def zbody(i, carry):
        for u in range(ZU):
            off = (i * ZU + u) * LANES
            for ar in arefs:
                ar[pl.ds(off, LANES)] = zero
        return carry

    lax.fori_loop(0, N // (LANES * ZU), zbody, 0)

    def _edge_copies(i, buf):
        off = i * CHUNK
        voff = buf * CHUNK
        return (
            pltpu.make_async_copy(
                src_hbm.at[pl.ds(off, CHUNK)], src_v.at[pl.ds(voff, CHUNK)], sem),
            pltpu.make_async_copy(
                dst_hbm.at[pl.ds(off, CHUNK)], dst_v.at[pl.ds(voff, CHUNK)], sem),
            pltpu.make_async_copy(
                ew_hbm.at[pl.ds(off, CHUNK)], ew_v.at[pl.ds(voff, CHUNK)], sem),
        )

    def _issue(i, buf):
        for cp in _edge_copies(i, buf):
            cp.start()

    _issue(0, 0)
    for cp in xcopies:
        cp.wait()

    def chunk_body(i, carry):
        buf = jnp.bitwise_and(i, 1)
        voff = buf * CHUNK
        for cp in _edge_copies(i, buf):
            cp.wait()

        @pl.when(i + 1 < NCH)
        def _prefetch():
            _issue(i + 1, 1 - buf)

        @plsc.parallel_loop(0, CHUNK // LANES, unroll=U)
        def gbody(g):
            base = voff + g * LANES
            s = src_v[pl.ds(base, LANES)]
            dd = dst_v[pl.ds(base, LANES)]
            w = ew_v[pl.ds(base, LANES)]
            for xr, ar in zip(xrefs, arefs):
                v = plsc.load_gather(xr, [s])
                plsc.addupdate_scatter(ar, [dd], v * w)

        return carry

    lax.fori_loop(0, NCH, chunk_body, 0)

    for c in range(CPW):
        pltpu.sync_copy(arefs[c], agg_hbm.at[pl.ds((c0 + c) * N, N)])


def _sc_agg(xT, src, dst, edge_weight):
    mesh = plsc.VectorSubcoreMesh(core_axis_name="c", subcore_axis_name="s")
    f = functools.partial(
        pl.kernel,
        mesh=mesh,
        compiler_params=pltpu.CompilerParams(needs_layout_passes=False),
        out_type=jax.ShapeDtypeStruct((D * N,), jnp.float32),
        scratch_types=[
            pltpu.VMEM((N,), jnp.float32),
            pltpu.VMEM((N,), jnp.float32),
            pltpu.VMEM((N,), jnp.float32),
            pltpu.VMEM((N,), jnp.float32),
            pltpu.VMEM((N,), jnp.float32),
            pltpu.VMEM((N,), jnp.float32),
            pltpu.VMEM((N,), jnp.float32),
            pltpu.VMEM((N,), jnp.float32),
            pltpu.VMEM((2 * CHUNK,), jnp.int32),
            pltpu.VMEM((2 * CHUNK,), jnp.int32),
            pltpu.VMEM((2 * CHUNK,), jnp.float32),
            pltpu.SemaphoreType.DMA,
            pltpu.SemaphoreType.DMA,
        ],
    )(_sc_agg_body)
    return f(xT.reshape(-1), src, dst, edge_weight).reshape(D, N)


def _mm_body(a_ref, w_ref, b_ref, o_ref):
    o_ref[...] = lax.dot_general(
        a_ref[...], w_ref[...], (((0,), (0,)), ((), ())),
        preferred_element_type=jnp.float32,
    ) + b_ref[...]


def _matmul_tc(aggT, W, b2):
    return pl.pallas_call(
        _mm_body,
        out_shape=jax.ShapeDtypeStruct((N, C), jnp.float32),
    )(aggT, W, b2)


def kernel(x, edge_index, edge_weight, W, b):
    xT = _transpose_tc(x)
    aggT = _sc_agg(xT, edge_index[0], edge_index[1], edge_weight)
    return _matmul_tc(aggT, W, b[None, :])


# speedup vs baseline: 7.8390x; 1.1004x over previous
import functools

import jax
import jax.numpy as jnp
from jax import lax
from jax.experimental import pallas as pl
from jax.experimental.pallas import tpu as pltpu
from jax.experimental.pallas import tpu_sc as plsc

N = 10000
E = 320000
D = 128
C = 128

NUM_CORES = 2
NUM_SUBCORES = 16
NW = NUM_CORES * NUM_SUBCORES
CPW = D // NW
CHUNK = 6400
NCH = E // CHUNK
U = 16
LANES = 16


def _tr_body(x_ref, ei_ref, o_ref, p_ref):
    o_ref[...] = x_ref[...].T
    p_ref[...] = ei_ref[0, :] | (ei_ref[1, :] << 16)


def _transpose_pack_tc(x, edge_index):
    return pl.pallas_call(
        _tr_body,
        out_shape=(
            jax.ShapeDtypeStruct((D, N), jnp.float32),
            jax.ShapeDtypeStruct((E,), jnp.int32),
        ),
    )(x, edge_index)


def _sc_agg_body(xT_hbm, pk_hbm, ew_hbm, agg_hbm, x0, x1, x2, x3,
                 a0, a1, a2, a3, pk_v, ew_v, sem, xsem):
    wid = lax.axis_index("s") * NUM_CORES + lax.axis_index("c")
    c0 = wid * CPW
    xrefs = (x0, x1, x2, x3)
    arefs = (a0, a1, a2, a3)

    xcopies = [
        pltpu.make_async_copy(xT_hbm.at[pl.ds((c0 + c) * N, N)], xrefs[c], xsem)
        for c in range(CPW)
    ]
    for cp in xcopies:
        cp.start()

    zero = jnp.zeros((LANES,), jnp.float32)
    ZU = 5

    def zbody(i, carry):
        for u in range(ZU):
            off = (i * ZU + u) * LANES
            for ar in arefs:
                ar[pl.ds(off, LANES)] = zero
        return carry

    lax.fori_loop(0, N // (LANES * ZU), zbody, 0)

    def _edge_copies(i, buf):
        off = i * CHUNK
        voff = buf * CHUNK
        return (
            pltpu.make_async_copy(
                pk_hbm.at[pl.ds(off, CHUNK)], pk_v.at[pl.ds(voff, CHUNK)], sem),
            pltpu.make_async_copy(
                ew_hbm.at[pl.ds(off, CHUNK)], ew_v.at[pl.ds(voff, CHUNK)], sem),
        )

    def _issue(i, buf):
        for cp in _edge_copies(i, buf):
            cp.start()

    _issue(0, 0)
    for cp in xcopies:
        cp.wait()

    def chunk_body(i, carry):
        buf = jnp.bitwise_and(i, 1)
        voff = buf * CHUNK
        for cp in _edge_copies(i, buf):
            cp.wait()

        @pl.when(i + 1 < NCH)
        def _prefetch():
            _issue(i + 1, 1 - buf)

        @plsc.parallel_loop(0, CHUNK // LANES, unroll=U)
        def gbody(g):
            base = voff + g * LANES
            p = pk_v[pl.ds(base, LANES)]
            s = p & 0xFFFF
            dd = lax.shift_right_logical(p, 16)
            w = ew_v[pl.ds(base, LANES)]
            for xr, ar in zip(xrefs, arefs):
                v = plsc.load_gather(xr, [s])
                plsc.addupdate_scatter(ar, [dd], v * w)

        return carry

    lax.fori_loop(0, NCH, chunk_body, 0)

    for c in range(CPW):
        pltpu.sync_copy(arefs[c], agg_hbm.at[pl.ds((c0 + c) * N, N)])


def _sc_agg(xT, packed, edge_weight):
    mesh = plsc.VectorSubcoreMesh(core_axis_name="c", subcore_axis_name="s")
    f = functools.partial(
        pl.kernel,
        mesh=mesh,
        compiler_params=pltpu.CompilerParams(needs_layout_passes=False),
        out_type=jax.ShapeDtypeStruct((D * N,), jnp.float32),
        scratch_types=[
            pltpu.VMEM((N,), jnp.float32),
            pltpu.VMEM((N,), jnp.float32),
            pltpu.VMEM((N,), jnp.float32),
            pltpu.VMEM((N,), jnp.float32),
            pltpu.VMEM((N,), jnp.float32),
            pltpu.VMEM((N,), jnp.float32),
            pltpu.VMEM((N,), jnp.float32),
            pltpu.VMEM((N,), jnp.float32),
            pltpu.VMEM((2 * CHUNK,), jnp.int32),
            pltpu.VMEM((2 * CHUNK,), jnp.float32),
            pltpu.SemaphoreType.DMA,
            pltpu.SemaphoreType.DMA,
        ],
    )(_sc_agg_body)
    return f(xT.reshape(-1), packed, edge_weight).reshape(D, N)


def _mm_body(a_ref, w_ref, b_ref, o_ref):
    o_ref[...] = lax.dot_general(
        a_ref[...], w_ref[...], (((0,), (0,)), ((), ())),
        preferred_element_type=jnp.float32,
    ) + b_ref[...]


def _matmul_tc(aggT, W, b2):
    return pl.pallas_call(
        _mm_body,
        out_shape=jax.ShapeDtypeStruct((N, C), jnp.float32),
    )(aggT, W, b2)


def kernel(x, edge_index, edge_weight, W, b):
    xT, packed = _transpose_pack_tc(x, edge_index)
    aggT = _sc_agg(xT, packed, edge_weight)
    return _matmul_tc(aggT, W, b[None, :])


# speedup vs baseline: 8.6594x; 1.1047x over previous
import functools

import jax
import jax.numpy as jnp
from jax import lax
from jax.experimental import pallas as pl
from jax.experimental.pallas import tpu as pltpu
from jax.experimental.pallas import tpu_sc as plsc

N = 10000
E = 320000
D = 128
C = 128

NUM_CORES = 2
NUM_SUBCORES = 16
NW = NUM_CORES * NUM_SUBCORES
CPW = D // NW
CHUNK = 6400
NCH = E // CHUNK
U = 16
LANES = 16


def _tr_body(x_ref, ei_ref, o_ref, p_ref):
    o_ref[...] = x_ref[...].T
    p_ref[...] = ei_ref[0, :] | (ei_ref[1, :] << 16)


def _transpose_pack_tc(x, edge_index):
    return pl.pallas_call(
        _tr_body,
        out_shape=(
            jax.ShapeDtypeStruct((D, N), jnp.float32),
            jax.ShapeDtypeStruct((E,), jnp.int32),
        ),
    )(x, edge_index)


def _sc_agg_body(xT_hbm, pk_hbm, ew_hbm, agg_hbm, xf0, xf1, xp0, xp1,
                 a0, a1, a2, a3, pk_v, ew_v, sem, xsem):
    wid = lax.axis_index("s") * NUM_CORES + lax.axis_index("c")
    c0 = wid * CPW
    arefs = (a0, a1, a2, a3)
    xprefs = (xp0, xp1)

    def _stage_pair(pair):
        cps = [
            pltpu.make_async_copy(
                xT_hbm.at[pl.ds((c0 + 2 * pair + k) * N, N)], (xf0, xf1)[k], xsem)
            for k in range(2)
        ]
        for cp in cps:
            cp.start()
        return cps

    def _pack_pair(pair):
        dstp = xprefs[pair]

        @plsc.parallel_loop(0, N // LANES, unroll=8)
        def pbody(i):
            off = i * LANES
            v0 = xf0[pl.ds(off, LANES)]
            v1 = xf1[pl.ds(off, LANES)]
            pk = plsc.pack(v0, v1, format=plsc.PackFormat.INTERLEAVED)
            dstp[pl.ds(off, LANES)] = plsc.bitcast(pk, jnp.int32)

    cps = _stage_pair(0)

    zero = jnp.zeros((LANES,), jnp.float32)
    ZU = 5

    def zbody(i, carry):
        for u in range(ZU):
            off = (i * ZU + u) * LANES
            for ar in arefs:
                ar[pl.ds(off, LANES)] = zero
        return carry

    lax.fori_loop(0, N // (LANES * ZU), zbody, 0)

    for cp in cps:
        cp.wait()
    _pack_pair(0)
    cps = _stage_pair(1)
    for cp in cps:
        cp.wait()
    _pack_pair(1)

    def _edge_copies(i, buf):
        off = i * CHUNK
        voff = buf * CHUNK
        return (
            pltpu.make_async_copy(
                pk_hbm.at[pl.ds(off, CHUNK)], pk_v.at[pl.ds(voff, CHUNK)], sem),
            pltpu.make_async_copy(
                ew_hbm.at[pl.ds(off, CHUNK)], ew_v.at[pl.ds(voff, CHUNK)], sem),
        )

    def _issue(i, buf):
        for cp in _edge_copies(i, buf):
            cp.start()

    _issue(0, 0)

    def chunk_body(i, carry):
        buf = jnp.bitwise_and(i, 1)
        voff = buf * CHUNK
        for cp in _edge_copies(i, buf):
            cp.wait()

        @pl.when(i + 1 < NCH)
        def _prefetch():
            _issue(i + 1, 1 - buf)

        @plsc.parallel_loop(0, CHUNK // LANES, unroll=U)
        def gbody(g):
            base = voff + g * LANES
            p = pk_v[pl.ds(base, LANES)]
            s = p & 0xFFFF
            dd = lax.shift_right_logical(p, 16)
            w = ew_v[pl.ds(base, LANES)]
            for j in range(2):
                g2 = plsc.load_gather(xprefs[j], [s])
                lo, hi = plsc.unpack(
                    plsc.bitcast(g2, jnp.bfloat16),
                    format=plsc.PackFormat.INTERLEAVED,
                    preferred_element_type=jnp.float32,
                )
                plsc.addupdate_scatter(arefs[2 * j], [dd], lo * w)
                plsc.addupdate_scatter(arefs[2 * j + 1], [dd], hi * w)

        return carry

    lax.fori_loop(0, NCH, chunk_body, 0)

    for c in range(CPW):
        pltpu.sync_copy(arefs[c], agg_hbm.at[pl.ds((c0 + c) * N, N)])


def _sc_agg(xT, packed, edge_weight):
    mesh = plsc.VectorSubcoreMesh(core_axis_name="c", subcore_axis_name="s")
    f = functools.partial(
        pl.kernel,
        mesh=mesh,
        compiler_params=pltpu.CompilerParams(needs_layout_passes=False),
        out_type=jax.ShapeDtypeStruct((D * N,), jnp.float32),
        scratch_types=[
            pltpu.VMEM((N,), jnp.float32),
            pltpu.VMEM((N,), jnp.float32),
            pltpu.VMEM((N,), jnp.int32),
            pltpu.VMEM((N,), jnp.int32),
            pltpu.VMEM((N,), jnp.float32),
            pltpu.VMEM((N,), jnp.float32),
            pltpu.VMEM((N,), jnp.float32),
            pltpu.VMEM((N,), jnp.float32),
            pltpu.VMEM((2 * CHUNK,), jnp.int32),
            pltpu.VMEM((2 * CHUNK,), jnp.float32),
            pltpu.SemaphoreType.DMA,
            pltpu.SemaphoreType.DMA,
        ],
    )(_sc_agg_body)
    return f(xT.reshape(-1), packed, edge_weight).reshape(D, N)


def _mm_body(a_ref, w_ref, b_ref, o_ref):
    o_ref[...] = lax.dot_general(
        a_ref[...], w_ref[...], (((0,), (0,)), ((), ())),
        preferred_element_type=jnp.float32,
    ) + b_ref[...]


def _matmul_tc(aggT, W, b2):
    return pl.pallas_call(
        _mm_body,
        out_shape=jax.ShapeDtypeStruct((N, C), jnp.float32),
    )(aggT, W, b2)


def kernel(x, edge_index, edge_weight, W, b):
    xT, packed = _transpose_pack_tc(x, edge_index)
    aggT = _sc_agg(xT, packed, edge_weight)
    return _matmul_tc(aggT, W, b[None, :])


# speedup vs baseline: 8.8145x; 1.0179x over previous
import functools

import jax
import jax.numpy as jnp
from jax import lax
from jax.experimental import pallas as pl
from jax.experimental.pallas import tpu as pltpu
from jax.experimental.pallas import tpu_sc as plsc

N = 10000
E = 320000
D = 128
C = 128

NUM_CORES = 2
NUM_SUBCORES = 16
NW = NUM_CORES * NUM_SUBCORES
CPW = D // NW
CHUNK = 8000
NCH = E // CHUNK
U = 10
LANES = 16


def _tr_body(x_ref, ei_ref, o_ref, p_ref):
    o_ref[...] = x_ref[...].T
    p_ref[...] = ei_ref[0, :] | (ei_ref[1, :] << 16)


def _transpose_pack_tc(x, edge_index):
    return pl.pallas_call(
        _tr_body,
        out_shape=(
            jax.ShapeDtypeStruct((D, N), jnp.float32),
            jax.ShapeDtypeStruct((E,), jnp.int32),
        ),
    )(x, edge_index)


def _sc_agg_body(xT_hbm, pk_hbm, ew_hbm, agg_hbm, xf0, xf1, xp0, xp1,
                 a0, a1, a2, a3, pk_v, ew_v, sem, xsem):
    wid = lax.axis_index("s") * NUM_CORES + lax.axis_index("c")
    c0 = wid * CPW
    arefs = (a0, a1, a2, a3)
    xprefs = (xp0, xp1)

    def _stage_pair(pair):
        cps = [
            pltpu.make_async_copy(
                xT_hbm.at[pl.ds((c0 + 2 * pair + k) * N, N)], (xf0, xf1)[k], xsem)
            for k in range(2)
        ]
        for cp in cps:
            cp.start()
        return cps

    def _pack_pair(pair):
        dstp = xprefs[pair]

        @plsc.parallel_loop(0, N // LANES, unroll=8)
        def pbody(i):
            off = i * LANES
            v0 = xf0[pl.ds(off, LANES)]
            v1 = xf1[pl.ds(off, LANES)]
            pk = plsc.pack(v0, v1, format=plsc.PackFormat.INTERLEAVED)
            dstp[pl.ds(off, LANES)] = plsc.bitcast(pk, jnp.int32)

    cps = _stage_pair(0)

    zero = jnp.zeros((LANES,), jnp.float32)
    ZU = 5

    def zbody(i, carry):
        for u in range(ZU):
            off = (i * ZU + u) * LANES
            for ar in arefs:
                ar[pl.ds(off, LANES)] = zero
        return carry

    lax.fori_loop(0, N // (LANES * ZU), zbody, 0)

    for cp in cps:
        cp.wait()
    _pack_pair(0)
    cps = _stage_pair(1)
    for cp in cps:
        cp.wait()
    _pack_pair(1)

    def _edge_copies(i, buf):
        off = i * CHUNK
        voff = buf * CHUNK
        return (
            pltpu.make_async_copy(
                pk_hbm.at[pl.ds(off, CHUNK)], pk_v.at[pl.ds(voff, CHUNK)], sem),
            pltpu.make_async_copy(
                ew_hbm.at[pl.ds(off, CHUNK)], ew_v.at[pl.ds(voff, CHUNK)], sem),
        )

    def _issue(i, buf):
        for cp in _edge_copies(i, buf):
            cp.start()

    _issue(0, 0)

    def chunk_body(i, carry):
        buf = jnp.bitwise_and(i, 1)
        voff = buf * CHUNK
        for cp in _edge_copies(i, buf):
            cp.wait()

        @pl.when(i + 1 < NCH)
        def _prefetch():
            _issue(i + 1, 1 - buf)

        @plsc.parallel_loop(0, CHUNK // LANES, unroll=U)
        def gbody(g):
            base = voff + g * LANES
            p = pk_v[pl.ds(base, LANES)]
            s = p & 0xFFFF
            dd = lax.shift_right_logical(p, 16)
            w = ew_v[pl.ds(base, LANES)]
            for j in range(2):
                g2 = plsc.load_gather(xprefs[j], [s])
                lo, hi = plsc.unpack(
                    plsc.bitcast(g2, jnp.bfloat16),
                    format=plsc.PackFormat.INTERLEAVED,
                    preferred_element_type=jnp.float32,
                )
                plsc.addupdate_scatter(arefs[2 * j], [dd], lo * w)
                plsc.addupdate_scatter(arefs[2 * j + 1], [dd], hi * w)

        return carry

    lax.fori_loop(0, NCH, chunk_body, 0)

    for c in range(CPW):
        pltpu.sync_copy(arefs[c], agg_hbm.at[pl.ds((c0 + c) * N, N)])


def _sc_agg(xT, packed, edge_weight):
    mesh = plsc.VectorSubcoreMesh(core_axis_name="c", subcore_axis_name="s")
    f = functools.partial(
        pl.kernel,
        mesh=mesh,
        compiler_params=pltpu.CompilerParams(needs_layout_passes=False),
        out_type=jax.ShapeDtypeStruct((D * N,), jnp.float32),
        scratch_types=[
            pltpu.VMEM((N,), jnp.float32),
            pltpu.VMEM((N,), jnp.float32),
            pltpu.VMEM((N,), jnp.int32),
            pltpu.VMEM((N,), jnp.int32),
            pltpu.VMEM((N,), jnp.float32),
            pltpu.VMEM((N,), jnp.float32),
            pltpu.VMEM((N,), jnp.float32),
            pltpu.VMEM((N,), jnp.float32),
            pltpu.VMEM((2 * CHUNK,), jnp.int32),
            pltpu.VMEM((2 * CHUNK,), jnp.float32),
            pltpu.SemaphoreType.DMA,
            pltpu.SemaphoreType.DMA,
        ],
    )(_sc_agg_body)
    return f(xT.reshape(-1), packed, edge_weight).reshape(D, N)


def _mm_body(a_ref, w_ref, b_ref, o_ref):
    o_ref[...] = lax.dot_general(
        a_ref[...], w_ref[...], (((0,), (0,)), ((), ())),
        preferred_element_type=jnp.float32,
    ) + b_ref[...]


def _matmul_tc(aggT, W, b2):
    return pl.pallas_call(
        _mm_body,
        out_shape=jax.ShapeDtypeStruct((N, C), jnp.float32),
    )(aggT, W, b2)


def kernel(x, edge_index, edge_weight, W, b):
    xT, packed = _transpose_pack_tc(x, edge_index)
    aggT = _sc_agg(xT, packed, edge_weight)
    return _matmul_tc(aggT, W, b[None, :])


# speedup vs baseline: 9.1705x; 1.0404x over previous
import functools

import jax
import jax.numpy as jnp
from jax import lax
from jax.experimental import pallas as pl
from jax.experimental.pallas import tpu as pltpu
from jax.experimental.pallas import tpu_sc as plsc

N = 10000
E = 320000
D = 128
C = 128

NUM_CORES = 2
NUM_SUBCORES = 16
NW = NUM_CORES * NUM_SUBCORES
CPW = D // NW
CHUNK = 12800
NCH = E // CHUNK
U = 10
LANES = 16


def _round_bf16_bits(v):
    r = (lax.shift_right_logical(v, 16) & 1) + 0x7FFF
    return lax.shift_right_logical(v + r, 16)


def _tr_body(x_ref, ei_ref, o_ref, p_ref):
    r2 = 2 * lax.broadcasted_iota(jnp.int32, (D // 2, D), 0)
    cc = lax.broadcasted_iota(jnp.int32, (D // 2, D), 1)
    pe = (cc == r2).astype(jnp.float32)
    po = (cc == r2 + 1).astype(jnp.float32)
    x = x_ref[...]
    dn = (((1,), (1,)), ((), ()))
    e_f = lax.dot_general(pe, x, dn, preferred_element_type=jnp.float32)
    o_f = lax.dot_general(po, x, dn, preferred_element_type=jnp.float32)
    e = _round_bf16_bits(lax.bitcast_convert_type(e_f, jnp.int32))
    o = _round_bf16_bits(lax.bitcast_convert_type(o_f, jnp.int32))
    o_ref[...] = e | (o << 16)
    p_ref[...] = ei_ref[0, :] | (ei_ref[1, :] << 16)


def _transpose_pack_tc(x, edge_index):
    return pl.pallas_call(
        _tr_body,
        out_shape=(
            jax.ShapeDtypeStruct((D // 2, N), jnp.int32),
            jax.ShapeDtypeStruct((E,), jnp.int32),
        ),
    )(x, edge_index)


def _sc_agg_body(xT_hbm, pk_hbm, ew_hbm, agg_hbm, xp0, xp1,
                 a0, a1, a2, a3, pk_v, ew_v, sem, xsem):
    wid = lax.axis_index("s") * NUM_CORES + lax.axis_index("c")
    c0 = wid * CPW
    arefs = (a0, a1, a2, a3)
    xprefs = (xp0, xp1)

    xcopies = [
        pltpu.make_async_copy(
            xT_hbm.at[pl.ds((2 * wid + j) * N, N)], xprefs[j], xsem)
        for j in range(2)
    ]
    for cp in xcopies:
        cp.start()

    zero = jnp.zeros((LANES,), jnp.float32)
    ZU = 5

    def zbody(i, carry):
        for u in range(ZU):
            off = (i * ZU + u) * LANES
            for ar in arefs:
                ar[pl.ds(off, LANES)] = zero
        return carry

    lax.fori_loop(0, N // (LANES * ZU), zbody, 0)

    for cp in xcopies:
        cp.wait()

    def _edge_copies(i, buf):
        off = i * CHUNK
        voff = buf * CHUNK
        return (
            pltpu.make_async_copy(
                pk_hbm.at[pl.ds(off, CHUNK)], pk_v.at[pl.ds(voff, CHUNK)], sem),
            pltpu.make_async_copy(
                ew_hbm.at[pl.ds(off, CHUNK)], ew_v.at[pl.ds(voff, CHUNK)], sem),
        )

    def _issue(i, buf):
        for cp in _edge_copies(i, buf):
            cp.start()

    _issue(0, 0)

    def chunk_body(i, carry):
        buf = jnp.bitwise_and(i, 1)
        voff = buf * CHUNK
        for cp in _edge_copies(i, buf):
            cp.wait()

        @pl.when(i + 1 < NCH)
        def _prefetch():
            _issue(i + 1, 1 - buf)

        @plsc.parallel_loop(0, CHUNK // LANES, unroll=U)
        def gbody(g):
            base = voff + g * LANES
            p = pk_v[pl.ds(base, LANES)]
            s = p & 0xFFFF
            dd = lax.shift_right_logical(p, 16)
            w = ew_v[pl.ds(base, LANES)]
            for j in range(2):
                g2 = plsc.load_gather(xprefs[j], [s])
                lo, hi = plsc.unpack(
                    plsc.bitcast(g2, jnp.bfloat16),
                    format=plsc.PackFormat.INTERLEAVED,
                    preferred_element_type=jnp.float32,
                )
                plsc.addupdate_scatter(arefs[2 * j], [dd], lo * w)
                plsc.addupdate_scatter(arefs[2 * j + 1], [dd], hi * w)

        return carry

    lax.fori_loop(0, NCH, chunk_body, 0)

    for c in range(CPW):
        pltpu.sync_copy(arefs[c], agg_hbm.at[pl.ds((c0 + c) * N, N)])


def _sc_agg(xT, packed, edge_weight):
    mesh = plsc.VectorSubcoreMesh(core_axis_name="c", subcore_axis_name="s")
    f = functools.partial(
        pl.kernel,
        mesh=mesh,
        compiler_params=pltpu.CompilerParams(needs_layout_passes=False),
        out_type=jax.ShapeDtypeStruct((D * N,), jnp.float32),
        scratch_types=[
            pltpu.VMEM((N,), jnp.int32),
            pltpu.VMEM((N,), jnp.int32),
            pltpu.VMEM((N,), jnp.float32),
            pltpu.VMEM((N,), jnp.float32),
            pltpu.VMEM((N,), jnp.float32),
            pltpu.VMEM((N,), jnp.float32),
            pltpu.VMEM((2 * CHUNK,), jnp.int32),
            pltpu.VMEM((2 * CHUNK,), jnp.float32),
            pltpu.SemaphoreType.DMA,
            pltpu.SemaphoreType.DMA,
        ],
    )(_sc_agg_body)
    return f(xT.reshape(-1), packed, edge_weight).reshape(D, N)


def _mm_body(a_ref, w_ref, b_ref, o_ref):
    o_ref[...] = lax.dot_general(
        a_ref[...], w_ref[...], (((0,), (0,)), ((), ())),
        preferred_element_type=jnp.float32,
    ) + b_ref[...]


def _matmul_tc(aggT, W, b2):
    return pl.pallas_call(
        _mm_body,
        out_shape=jax.ShapeDtypeStruct((N, C), jnp.float32),
    )(aggT, W, b2)


def kernel(x, edge_index, edge_weight, W, b):
    xT, packed = _transpose_pack_tc(x, edge_index)
    aggT = _sc_agg(xT, packed, edge_weight)
    return _matmul_tc(aggT, W, b[None, :])
